# baseline (speedup 1.0000x reference)
import functools
import jax
import jax.numpy as jnp
from jax import lax
from jax.experimental import pallas as pl
from jax.experimental.pallas import tpu as pltpu
from jax.experimental.pallas import tpu_sc as plsc

VOCAB = 1000
VPAD = 1024
NBLK = VPAD // 128
NC, NS = 2, 16
NW = NC * NS
CHUNK = 32
NBUF = 3


def _body(n_chunks, idx_hbm, table_hbm, out_hbm, idx_v,
          ic0, ic1, ic2, buf0, buf1, buf2,
          gs0, gs1, gs2, ss0, ss1, ss2):
    ics = (ic0, ic1, ic2)
    bufs = (buf0, buf1, buf2)
    gss = (gs0, gs1, gs2)
    sss = (ss0, ss1, ss2)
    wid = lax.axis_index("s") * NC + lax.axis_index("c")
    pltpu.sync_copy(idx_hbm.at[wid], idx_v)
    base = wid * (n_chunks * CHUNK)

    def compute_idx(c, j):
        for k in range(CHUNK // 16):
            v = idx_v[pl.ds(c * CHUNK + 16 * k, 16)] * NBLK
            for C in range(NBLK):
                ics[j][pl.ds(C * CHUNK + 16 * k, 16)] = v + C

    def g_descs(j):
        return [(table_hbm.at[ics[j].at[pl.ds(C * CHUNK, CHUNK)]],
                 bufs[j].at[:, pl.ds(128 * C, 128)], gss[j])
                for C in range(NBLK)]

    def g_start(j):
        for s, d, sm in g_descs(j):
            pltpu.async_copy(s, d, sm)

    def g_wait(j):
        for s, d, sm in g_descs(j):
            pltpu.make_async_copy(s, d, sm).wait()

    def out_slice(c):
        return out_hbm.at[pl.ds(base + c * CHUNK, CHUNK)]

    def s_start(c, j):
        pltpu.async_copy(bufs[j], out_slice(c), sss[j])

    def s_wait(c, j):
        pltpu.make_async_copy(bufs[j], out_slice(c), sss[j]).wait()

    for j in range(NBUF):
        compute_idx(j, j)
        g_start(j)

    n_main = n_chunks // NBUF - 1

    def it(i, carry):
        c = NBUF * i
        for j in range(NBUF):
            g_wait(j)
            s_start(c + j, j)
        for j in range(NBUF):
            s_wait(c + j, j)
            compute_idx(c + j + NBUF, j)
            g_start(j)
        return carry

    lax.fori_loop(0, n_main, it, 0)

    c = n_main * NBUF
    rem = n_chunks - c
    for j in range(NBUF):
        g_wait(j)
        s_start(c + j, j)
    for j in range(rem - NBUF):
        s_wait(c + j, j)
        compute_idx(c + j + NBUF, j)
        g_start(j)
    for j in range(rem - NBUF, NBUF):
        s_wait(c + j, j)
    for j in range(rem - NBUF):
        g_wait(j)
        s_start(c + j + NBUF, j)
        s_wait(c + j + NBUF, j)


def kernel(context, table):
    b, t = context.shape
    n = b * t
    assert n % (NW * CHUNK) == 0
    n_chunks = n // (NW * CHUNK)
    assert n_chunks >= 2 * NBUF
    idx = context.reshape(NW, n_chunks * CHUNK).astype(jnp.int32)
    table_r = jnp.pad(table, ((0, 0), (0, VPAD - VOCAB))).reshape(VOCAB * NBLK, 128)

    mesh = plsc.VectorSubcoreMesh(core_axis_name="c", subcore_axis_name="s")
    run = pl.kernel(
        functools.partial(_body, n_chunks),
        out_type=jax.ShapeDtypeStruct((n, VPAD), jnp.float32),
        mesh=mesh,
        scratch_types=(
            [pltpu.VMEM((n_chunks * CHUNK,), jnp.int32)]
            + [pltpu.VMEM((NBLK * CHUNK,), jnp.int32)] * NBUF
            + [pltpu.VMEM((CHUNK, VPAD), jnp.float32)] * NBUF
            + [pltpu.SemaphoreType.DMA] * (2 * NBUF)
        ),
    )
    out = run(idx, table_r)
    return out[:, :VOCAB].reshape(b, t, VOCAB)
